# speedup vs baseline: 1.7939x; 1.7939x over previous
import math

import jax
import jax.numpy as jnp
import numpy as np
from jax.experimental import pallas as pl

S = 2048
H = 768
NH = 12
NKV = 4
HD = 64
HALF = HD // 2
I = 768
E = 64
THETA = 150000.0
EPS = 1e-6
BLK = 256
GRID = S // BLK
REP = NH // NKV
Q_SIZE = NH * HD
KV_SIZE = NKV * HD


def _rope_perm() -> np.ndarray:
    perm = []
    for half in range(2):
        for h in range(NH):
            for j in range(HALF):
                perm.append(h * HD + half * HALF + j)
    for half in range(2):
        for g in range(NKV):
            for j in range(HALF):
                perm.append(Q_SIZE + g * HD + half * HALF + j)
    perm.extend(range(Q_SIZE + KV_SIZE, Q_SIZE + 2 * KV_SIZE))
    return np.asarray(perm, dtype=np.int32)


_PERM = _rope_perm()
_NEG = -1e30


def _stage1_body(pos_ref, hid_ref, res_ref, w_ref, b_ref, ln_ref,
                 q_out, k_out, v_out, r1_out):
    x = hid_ref[...] + res_ref[...]
    r1_out[...] = x
    ms = jnp.mean(x * x, axis=1, keepdims=True)
    h = x * jax.lax.rsqrt(ms + EPS) * ln_ref[...]
    qkv = jnp.dot(h.astype(jnp.bfloat16), w_ref[...],
                  preferred_element_type=jnp.float32) + b_ref[...]

    pos = pos_ref[...]
    jq = jax.lax.rem(jax.lax.broadcasted_iota(jnp.int32, (1, NH * HALF), 1),
                     HALF).astype(jnp.float32)
    inv_freq = jnp.exp(jq * (-math.log(THETA) / HALF))
    f = pos * inv_freq
    cos_q = jnp.cos(f)
    sin_q = jnp.sin(f)
    cos_k = cos_q[:, :NKV * HALF]
    sin_k = sin_q[:, :NKV * HALF]

    q1 = qkv[:, :NH * HALF]
    q2 = qkv[:, NH * HALF:Q_SIZE]
    k1 = qkv[:, Q_SIZE:Q_SIZE + NKV * HALF]
    k2 = qkv[:, Q_SIZE + NKV * HALF:Q_SIZE + KV_SIZE]
    v = qkv[:, Q_SIZE + KV_SIZE:]

    q_out[...] = jnp.concatenate(
        [q1 * cos_q - q2 * sin_q, q2 * cos_q + q1 * sin_q],
        axis=1).astype(jnp.bfloat16)
    k_out[...] = jnp.concatenate(
        [k1 * cos_k - k2 * sin_k, k2 * cos_k + k1 * sin_k],
        axis=1).astype(jnp.bfloat16)
    v_out[...] = v.astype(jnp.bfloat16)


def _stage2_body(q_ref, k_ref, v_ref, r1_ref, wo_ref, bo_ref, ln2_ref,
                 wr_ref, br_ref, wgu_ref, bgu_ref, wd_ref, bd_ref,
                 out_ref, r2_out):
    i = pl.program_id(0)
    q0 = i * BLK
    scale = HD ** -0.5

    row = jax.lax.rem(
        jax.lax.broadcasted_iota(jnp.int32, (REP * BLK, 1), 0), BLK)
    col = jax.lax.broadcasted_iota(jnp.int32, (1, S), 1)
    mask = col <= (q0 + row)

    o_cols = []
    for g in range(NKV):
        k_g = jnp.concatenate(
            [k_ref[:, g * HALF:(g + 1) * HALF],
             k_ref[:, NKV * HALF + g * HALF:NKV * HALF + (g + 1) * HALF]],
            axis=1)
        v_g = v_ref[:, g * HD:(g + 1) * HD]
        qs = []
        for hh in range(REP):
            h = g * REP + hh
            qs.append(jnp.concatenate(
                [q_ref[:, h * HALF:(h + 1) * HALF],
                 q_ref[:, NH * HALF + h * HALF:NH * HALF + (h + 1) * HALF]],
                axis=1))
        q_g = jnp.concatenate(qs, axis=0)
        s = jax.lax.dot_general(
            q_g, k_g, (((1,), (1,)), ((), ())),
            preferred_element_type=jnp.float32) * scale
        s = jnp.where(mask, s, _NEG)
        m = jnp.max(s, axis=1, keepdims=True)
        p = jnp.exp(s - m)
        l = jnp.sum(p, axis=1, keepdims=True)
        o_g = jnp.dot(p.astype(jnp.bfloat16), v_g,
                      preferred_element_type=jnp.float32) / l
        for hh in range(REP):
            o_cols.append(o_g[hh * BLK:(hh + 1) * BLK, :])
    o = jnp.concatenate(o_cols, axis=1).astype(jnp.bfloat16)

    attn = jnp.dot(o, wo_ref[...],
                   preferred_element_type=jnp.float32) + bo_ref[...]
    r2 = attn + r1_ref[...]
    r2_out[...] = r2

    ms = jnp.mean(r2 * r2, axis=1, keepdims=True)
    h2 = (r2 * jax.lax.rsqrt(ms + EPS) * ln2_ref[...]).astype(jnp.bfloat16)

    logits = jnp.dot(h2, wr_ref[...],
                     preferred_element_type=jnp.float32) + br_ref[...]
    m1 = jnp.max(logits, axis=1, keepdims=True)
    s2 = jnp.max(jnp.where(logits >= m1, _NEG, logits), axis=1, keepdims=True)
    e2 = jnp.exp(s2 - m1)
    denom = 1.0 + e2
    factor = 1.0 / denom + e2 / denom

    gu = jnp.dot(h2, wgu_ref[...],
                 preferred_element_type=jnp.float32) + bgu_ref[...]
    gate = gu[:, :I]
    up = gu[:, I:]
    x = gate * (up * jax.nn.sigmoid(up))
    eo = jnp.dot(x.astype(jnp.bfloat16), wd_ref[...],
                 preferred_element_type=jnp.float32) + bd_ref[...]
    out_ref[...] = factor * eo


def kernel(positions, hidden_states, residual, w_qkv, b_qkv, w_o, b_o,
           w_router, b_router, w_gate_up, b_gate_up, w_down, b_down,
           ln1_w, ln2_w):
    f32 = jnp.float32
    bf16 = jnp.bfloat16
    pos = positions.astype(f32).reshape(S, 1)
    w_qkv_t = w_qkv.T[:, _PERM].astype(bf16)
    b_qkv_p = b_qkv[_PERM].reshape(1, -1).astype(f32)
    wo_t = w_o.T.astype(bf16)
    wr_t = w_router.T.astype(bf16)
    wgu_t = w_gate_up.T.astype(bf16)
    wd_t = w_down.T.astype(bf16)

    full = lambda shape: pl.BlockSpec(shape, lambda i: (0, 0))
    blk = lambda cols: pl.BlockSpec((BLK, cols), lambda i: (i, 0))

    q_ro, k_ro, v, r1 = pl.pallas_call(
        _stage1_body,
        grid=(GRID,),
        in_specs=[
            blk(1),
            blk(H),
            blk(H),
            full((H, Q_SIZE + 2 * KV_SIZE)),
            full((1, Q_SIZE + 2 * KV_SIZE)),
            full((1, H)),
        ],
        out_specs=[blk(Q_SIZE), blk(KV_SIZE), blk(KV_SIZE), blk(H)],
        out_shape=[
            jax.ShapeDtypeStruct((S, Q_SIZE), bf16),
            jax.ShapeDtypeStruct((S, KV_SIZE), bf16),
            jax.ShapeDtypeStruct((S, KV_SIZE), bf16),
            jax.ShapeDtypeStruct((S, H), f32),
        ],
    )(pos, hidden_states, residual, w_qkv_t, b_qkv_p,
      ln1_w.reshape(1, H).astype(f32))

    out, r2 = pl.pallas_call(
        _stage2_body,
        grid=(GRID,),
        in_specs=[
            blk(Q_SIZE),
            full((S, KV_SIZE)),
            full((S, KV_SIZE)),
            blk(H),
            full((Q_SIZE, H)),
            full((1, H)),
            full((1, H)),
            full((H, E)),
            full((1, E)),
            full((H, 2 * I)),
            full((1, 2 * I)),
            full((I, H)),
            full((1, H)),
        ],
        out_specs=[blk(H), blk(H)],
        out_shape=[
            jax.ShapeDtypeStruct((S, H), f32),
            jax.ShapeDtypeStruct((S, H), f32),
        ],
    )(q_ro, k_ro, v, r1,
      wo_t, b_o.reshape(1, H).astype(f32), ln2_w.reshape(1, H).astype(f32),
      wr_t, b_router.reshape(1, E).astype(f32),
      wgu_t, b_gate_up.reshape(1, 2 * I).astype(f32),
      wd_t, b_down.reshape(1, H).astype(f32))

    return (out, r2)
